# baseline (speedup 1.0000x reference)
import functools

import jax
import jax.numpy as jnp
from jax import lax
from jax.experimental import pallas as pl
from jax.experimental.pallas import tpu as pltpu
from jax.experimental.pallas import tpu_sc as plsc

_NC = 2
_NS = 16


def kernel(hidden_states, position_embeddings):
    B, S, D = hidden_states.shape
    assert position_embeddings.shape == (S, D)
    NW = _NC * _NS
    rows_per_w = S // NW
    mesh = plsc.VectorSubcoreMesh(core_axis_name="c", subcore_axis_name="s")

    @functools.partial(
        pl.kernel,
        mesh=mesh,
        out_type=jax.ShapeDtypeStruct((B, S, D), jnp.float32),
        scratch_types=[pltpu.SemaphoreType.DMA],
    )
    def sc_bcast(table_hbm, out_hbm, sem):
        wid = lax.axis_index("s") * _NC + lax.axis_index("c")
        base = wid * rows_per_w
        cps = [
            pltpu.make_async_copy(
                table_hbm.at[pl.ds(base, rows_per_w)],
                out_hbm.at[b, pl.ds(base, rows_per_w)],
                sem,
            )
            for b in range(B)
        ]
        for c in cps:
            c.start()
        for c in cps:
            c.wait()

    return sc_bcast(position_embeddings)


# speedup vs baseline: 55.1411x; 55.1411x over previous
import functools

import jax
import jax.numpy as jnp
from jax import lax
from jax.experimental import pallas as pl
from jax.experimental.pallas import tpu as pltpu
from jax.experimental.pallas import tpu_sc as plsc

_NC = 2
_NS = 16
_NBUF = 3


def kernel(hidden_states, position_embeddings):
    B, S, D = hidden_states.shape
    assert position_embeddings.shape == (S, D)
    NW = _NC * _NS
    rows_per_w = S // NW
    CH = 32
    n_ch = rows_per_w // CH
    mesh = plsc.VectorSubcoreMesh(core_axis_name="c", subcore_axis_name="s")

    @functools.partial(
        pl.kernel,
        mesh=mesh,
        out_type=jax.ShapeDtypeStruct((B, S, D), jnp.float32),
        scratch_types=[
            pltpu.VMEM((CH, D), jnp.float32),
            pltpu.VMEM((CH, D), jnp.float32),
            pltpu.VMEM((CH, D), jnp.float32),
            pltpu.SemaphoreType.DMA,
            pltpu.SemaphoreType.DMA,
        ],
    )
    def sc_bcast(table_hbm, out_hbm, buf0, buf1, buf2, rsem, wsem):
        wid = lax.axis_index("s") * _NC + lax.axis_index("c")
        base = wid * rows_per_w
        bufs = (buf0, buf1, buf2)

        def rd(i):
            return pltpu.make_async_copy(
                table_hbm.at[pl.ds(base + i * CH, CH)], bufs[i % _NBUF], rsem
            )

        def wr(i, b):
            return pltpu.make_async_copy(
                bufs[i % _NBUF], out_hbm.at[b, pl.ds(base + i * CH, CH)], wsem
            )

        rd(0).start()
        rd(1).start()
        for i in range(n_ch):
            rd(i).wait()
            if i + 2 < n_ch:
                if i >= 1:
                    for b in range(B):
                        wr(i - 1, b).wait()
                rd(i + 2).start()
            for b in range(B):
                wr(i, b).start()
        for i in (n_ch - 3, n_ch - 2, n_ch - 1):
            for b in range(B):
                wr(i, b).wait()

    return sc_bcast(position_embeddings)
